# Initial kernel scaffold; baseline (speedup 1.0000x reference)
import jax
import jax.numpy as jnp
from jax import lax
from jax.experimental import pallas as pl
from jax.experimental.pallas import tpu as pltpu
from jax.experimental.pallas import tpu_sc as plsc

N = 10000
E = 320000
D = 128
NC = 2
NS = 16
NW = NC * NS
EPT = E // NW
CHUNK = 80
NCHUNKS = EPT // CHUNK
RPT = N // NS

f32 = jnp.float32


def _make_sc_aggregate(with_cnt: bool):
  out_type = [jax.ShapeDtypeStruct((NC, N, D), f32)]
  if with_cnt:
    out_type.append(jax.ShapeDtypeStruct((NC, N, 16), f32))

  scratch = [
      pltpu.VMEM((NCHUNKS, CHUNK), jnp.int32),
      pltpu.VMEM((NCHUNKS, CHUNK), jnp.int32),
      pltpu.VMEM((CHUNK, D), f32),
      pltpu.VMEM((CHUNK, 16), f32),
      pltpu.VMEM_SHARED((N, D), f32),
      pltpu.VMEM_SHARED((N, 16), f32),
      pltpu.SemaphoreType.DMA,
  ]
  mesh = plsc.VectorSubcoreMesh(core_axis_name="c", subcore_axis_name="s")

  def body(y_hbm, src_hbm, dst_hbm, zrow_hbm, z16_hbm, *rest):
    if with_cnt:
      msum_hbm, cnt_hbm = rest[0], rest[1]
      rest = rest[2:]
    else:
      msum_hbm = rest[0]
      rest = rest[1:]
    srcidx, dstidx, rows, ones_v, acc, cnt_acc, gsem = rest

    cid = lax.axis_index("c")
    sid = lax.axis_index("s")
    zbase = sid * RPT

    pltpu.sync_copy(zrow_hbm, acc.at[pl.ds(zbase, RPT)])
    if with_cnt:
      pltpu.sync_copy(z16_hbm, cnt_acc.at[pl.ds(zbase, RPT)])
    pltpu.sync_copy(src_hbm.at[cid, sid], srcidx)
    pltpu.sync_copy(dst_hbm.at[cid, sid], dstidx)
    if with_cnt:
      def fill(i, c):
        ones_v[i] = jnp.ones((16,), f32)
        return c
      lax.fori_loop(0, CHUNK, fill, 0)
    plsc.subcore_barrier()

    def step(i, c):
      pltpu.async_copy(y_hbm.at[srcidx.at[i]], rows, gsem).wait()
      pltpu.sync_copy(rows, acc.at[dstidx.at[i]], add=True)
      if with_cnt:
        pltpu.sync_copy(ones_v, cnt_acc.at[dstidx.at[i]], add=True)
      return c
    lax.fori_loop(0, NCHUNKS, step, 0)
    plsc.subcore_barrier()

    pltpu.sync_copy(acc.at[pl.ds(zbase, RPT)],
                    msum_hbm.at[cid, pl.ds(zbase, RPT)])
    if with_cnt:
      pltpu.sync_copy(cnt_acc.at[pl.ds(zbase, RPT)],
                      cnt_hbm.at[cid, pl.ds(zbase, RPT)])

  return pl.kernel(body, out_type=out_type, mesh=mesh, scratch_types=scratch)


_sc_agg_cnt = _make_sc_aggregate(with_cnt=True)
_sc_agg = _make_sc_aggregate(with_cnt=False)


BLK = 2000


def _proj_body(x_ref, wl_ref, wr_ref, b_ref, y_ref, z_ref):
  xb = x_ref[...]
  y_ref[...] = jnp.dot(xb, wl_ref[...], preferred_element_type=f32)
  z_ref[...] = jnp.dot(xb, wr_ref[...], preferred_element_type=f32) + b_ref[...]


def _stage_first(x, wl, wr, b):
  return pl.pallas_call(
      _proj_body,
      grid=(N // BLK,),
      in_specs=[
          pl.BlockSpec((BLK, D), lambda i: (i, 0)),
          pl.BlockSpec((D, D), lambda i: (0, 0)),
          pl.BlockSpec((D, D), lambda i: (0, 0)),
          pl.BlockSpec((1, D), lambda i: (0, 0)),
      ],
      out_specs=[pl.BlockSpec((BLK, D), lambda i: (i, 0))] * 2,
      out_shape=[jax.ShapeDtypeStruct((N, D), f32)] * 2,
  )(x, wl, wr, b.reshape(1, D))


def _mid_body(msum_ref, cnt_ref, z_ref, wl_ref, wr_ref, b_ref, y_ref, z2_ref):
  c = jnp.maximum(cnt_ref[0, :, :1] + cnt_ref[1, :, :1], 1.0)
  h = (msum_ref[0] + msum_ref[1]) / c + z_ref[...]
  h = jnp.maximum(h, 0.0)
  y_ref[...] = jnp.dot(h, wl_ref[...], preferred_element_type=f32)
  z2_ref[...] = jnp.dot(h, wr_ref[...], preferred_element_type=f32) + b_ref[...]


def _stage_mid(msum, cnt, z, wl, wr, b):
  return pl.pallas_call(
      _mid_body,
      grid=(N // BLK,),
      in_specs=[
          pl.BlockSpec((NC, BLK, D), lambda i: (0, i, 0)),
          pl.BlockSpec((NC, BLK, 16), lambda i: (0, i, 0)),
          pl.BlockSpec((BLK, D), lambda i: (i, 0)),
          pl.BlockSpec((D, D), lambda i: (0, 0)),
          pl.BlockSpec((D, D), lambda i: (0, 0)),
          pl.BlockSpec((1, D), lambda i: (0, 0)),
      ],
      out_specs=[pl.BlockSpec((BLK, D), lambda i: (i, 0))] * 2,
      out_shape=[jax.ShapeDtypeStruct((N, D), f32)] * 2,
  )(msum, cnt, z, wl, wr, b.reshape(1, D))


def _final_body(msum_ref, cnt_ref, z_ref, out_ref):
  c = jnp.maximum(cnt_ref[0, :, :1] + cnt_ref[1, :, :1], 1.0)
  out_ref[...] = (msum_ref[0] + msum_ref[1]) / c + z_ref[...]


def _stage_final(msum, cnt, z):
  return pl.pallas_call(
      _final_body,
      grid=(N // BLK,),
      in_specs=[
          pl.BlockSpec((NC, BLK, D), lambda i: (0, i, 0)),
          pl.BlockSpec((NC, BLK, 16), lambda i: (0, i, 0)),
          pl.BlockSpec((BLK, D), lambda i: (i, 0)),
      ],
      out_specs=pl.BlockSpec((BLK, D), lambda i: (i, 0)),
      out_shape=jax.ShapeDtypeStruct((N, D), f32),
  )(msum, cnt, z)


def kernel(x, edge_index, W1l, b1, W1r, W2l, b2, W2r):
  src = edge_index[0].astype(jnp.int32).reshape(NC, NS, NCHUNKS, CHUNK)
  dst = edge_index[1].astype(jnp.int32).reshape(NC, NS, NCHUNKS, CHUNK)
  zrow = jnp.zeros((RPT, D), f32)
  z16 = jnp.zeros((RPT, 16), f32)

  y1, z1 = _stage_first(x, W1l, W1r, b1)
  msum1, cnt = _sc_agg_cnt(y1, src, dst, zrow, z16)
  y2, z2 = _stage_mid(msum1, cnt, z1, W2l, W2r, b2)
  msum2, = _sc_agg(y2, src, dst, zrow, z16)
  return _stage_final(msum2, cnt, z2)


# speedup vs baseline: 4.5347x; 4.5347x over previous
import jax
import jax.numpy as jnp
from jax import lax
from jax.experimental import pallas as pl
from jax.experimental.pallas import tpu as pltpu
from jax.experimental.pallas import tpu_sc as plsc

N = 10000
E = 320000
D = 128
NC = 2
NS = 16
NW = NC * NS
EPT = E // NW
CHUNK = 80
NCHUNKS = EPT // CHUNK
BROW = 125
NBLK = N // BROW
BPT = NBLK // NS

f32 = jnp.float32


def _make_sc_aggregate(with_cnt: bool):
  out_type = [jax.ShapeDtypeStruct((NC, NBLK, BROW, D), f32)]

  scratch = [
      pltpu.VMEM((CHUNK,), jnp.int32),
      pltpu.VMEM((CHUNK,), jnp.int32),
      pltpu.VMEM((1, BROW), jnp.int32),
      pltpu.VMEM((BROW, D), f32),
      pltpu.VMEM_SHARED((N, D), f32),
      pltpu.SemaphoreType.DMA,
  ]
  mesh = plsc.VectorSubcoreMesh(core_axis_name="c", subcore_axis_name="s")
  ones_mode = with_cnt

  def body(y_hbm, src_hbm, dst_hbm, zrow_hbm, orow_hbm, ident_hbm, msum_hbm,
           srcbuf, dstbuf, identbuf, stage, acc, gsem):
    cid = lax.axis_index("c")
    sid = lax.axis_index("s")
    tbase = (cid * NS + sid) * EPT

    pltpu.sync_copy(zrow_hbm, stage)

    def zero_blk(k, c):
      blk = sid * BPT + k
      pltpu.sync_copy(ident_hbm.at[blk], identbuf)
      pltpu.sync_copy(stage, acc.at[identbuf.at[0]])
      return c
    lax.fori_loop(0, BPT, zero_blk, 0)

    if ones_mode:
      pltpu.sync_copy(orow_hbm, stage)
    plsc.subcore_barrier()

    def step(i, c):
      ebase = tbase + i * CHUNK
      pltpu.sync_copy(dst_hbm.at[pl.ds(ebase, CHUNK)], dstbuf)
      if not ones_mode:
        pltpu.sync_copy(src_hbm.at[pl.ds(ebase, CHUNK)], srcbuf)
        pltpu.async_copy(y_hbm.at[srcbuf], stage.at[pl.ds(0, CHUNK)],
                         gsem).wait()
      pltpu.sync_copy(stage.at[pl.ds(0, CHUNK)], acc.at[dstbuf], add=True)
      return c
    lax.fori_loop(0, NCHUNKS, step, 0)
    plsc.subcore_barrier()

    def drain_blk(k, c):
      blk = sid * BPT + k
      pltpu.sync_copy(ident_hbm.at[blk], identbuf)
      pltpu.async_copy(acc.at[identbuf.at[0]], stage, gsem).wait()
      pltpu.sync_copy(stage, msum_hbm.at[cid, blk])
      return c
    lax.fori_loop(0, BPT, drain_blk, 0)

  return pl.kernel(body, out_type=out_type, mesh=mesh, scratch_types=scratch)


_sc_count = _make_sc_aggregate(with_cnt=True)
_sc_agg = _make_sc_aggregate(with_cnt=False)


BLK = 2000


def _proj_body(x_ref, wl_ref, wr_ref, b_ref, y_ref, z_ref):
  xb = x_ref[...]
  y_ref[...] = jnp.dot(xb, wl_ref[...], preferred_element_type=f32)
  z_ref[...] = jnp.dot(xb, wr_ref[...], preferred_element_type=f32) + b_ref[...]


def _stage_first(x, wl, wr, b):
  return pl.pallas_call(
      _proj_body,
      grid=(N // BLK,),
      in_specs=[
          pl.BlockSpec((BLK, D), lambda i: (i, 0)),
          pl.BlockSpec((D, D), lambda i: (0, 0)),
          pl.BlockSpec((D, D), lambda i: (0, 0)),
          pl.BlockSpec((1, D), lambda i: (0, 0)),
      ],
      out_specs=[pl.BlockSpec((BLK, D), lambda i: (i, 0))] * 2,
      out_shape=[jax.ShapeDtypeStruct((N, D), f32)] * 2,
  )(x, wl, wr, b.reshape(1, D))


def _mid_body(msum_ref, cnt_ref, z_ref, wl_ref, wr_ref, b_ref, y_ref, z2_ref):
  c = jnp.maximum(cnt_ref[0, :, :1] + cnt_ref[1, :, :1], 1.0)
  h = (msum_ref[0] + msum_ref[1]) / c + z_ref[...]
  h = jnp.maximum(h, 0.0)
  y_ref[...] = jnp.dot(h, wl_ref[...], preferred_element_type=f32)
  z2_ref[...] = jnp.dot(h, wr_ref[...], preferred_element_type=f32) + b_ref[...]


def _stage_mid(msum, cnt, z, wl, wr, b):
  return pl.pallas_call(
      _mid_body,
      grid=(N // BLK,),
      in_specs=[
          pl.BlockSpec((NC, BLK, D), lambda i: (0, i, 0)),
          pl.BlockSpec((NC, BLK, D), lambda i: (0, i, 0)),
          pl.BlockSpec((BLK, D), lambda i: (i, 0)),
          pl.BlockSpec((D, D), lambda i: (0, 0)),
          pl.BlockSpec((D, D), lambda i: (0, 0)),
          pl.BlockSpec((1, D), lambda i: (0, 0)),
      ],
      out_specs=[pl.BlockSpec((BLK, D), lambda i: (i, 0))] * 2,
      out_shape=[jax.ShapeDtypeStruct((N, D), f32)] * 2,
  )(msum, cnt, z, wl, wr, b.reshape(1, D))


def _final_body(msum_ref, cnt_ref, z_ref, out_ref):
  c = jnp.maximum(cnt_ref[0, :, :1] + cnt_ref[1, :, :1], 1.0)
  out_ref[...] = (msum_ref[0] + msum_ref[1]) / c + z_ref[...]


def _stage_final(msum, cnt, z):
  return pl.pallas_call(
      _final_body,
      grid=(N // BLK,),
      in_specs=[
          pl.BlockSpec((NC, BLK, D), lambda i: (0, i, 0)),
          pl.BlockSpec((NC, BLK, D), lambda i: (0, i, 0)),
          pl.BlockSpec((BLK, D), lambda i: (i, 0)),
      ],
      out_specs=pl.BlockSpec((BLK, D), lambda i: (i, 0)),
      out_shape=jax.ShapeDtypeStruct((N, D), f32),
  )(msum, cnt, z)


def kernel(x, edge_index, W1l, b1, W1r, W2l, b2, W2r):
  src = edge_index[0].astype(jnp.int32)
  dst = edge_index[1].astype(jnp.int32)
  zrow = jnp.zeros((BROW, D), f32)
  orow = jnp.ones((BROW, D), f32)
  ident = jnp.arange(N, dtype=jnp.int32).reshape(NBLK, 1, BROW)

  y1, z1 = _stage_first(x, W1l, W1r, b1)
  cnt, = _sc_count(y1, src, dst, zrow, orow, ident)
  cnt = cnt.reshape(NC, N, D)
  msum1, = _sc_agg(y1, src, dst, zrow, orow, ident)
  msum1 = msum1.reshape(NC, N, D)
  y2, z2 = _stage_mid(msum1, cnt, z1, W2l, W2r, b2)
  msum2, = _sc_agg(y2, src, dst, zrow, orow, ident)
  msum2 = msum2.reshape(NC, N, D)
  return _stage_final(msum2, cnt, z2)
